# baseline (speedup 1.0000x reference)
import jax
import jax.numpy as jnp
from jax import lax
from jax.experimental import pallas as pl
from jax.experimental.pallas import tpu as pltpu
from jax.experimental.pallas import tpu_sc as plsc

NUM_FIELDS = 26
VOCAB = 100000
EMBED_DIM = 16
BATCH = 16384
NUM_PAIRS = (NUM_FIELDS * (NUM_FIELDS - 1)) // 2

_INFO = plsc.get_sparse_core_info()
NC = _INFO.num_cores
NS = _INFO.num_subcores
NW = NC * NS
LANES = _INFO.num_lanes

CHUNK = 64
PER_W = BATCH // NW
N_ITERS = PER_W // CHUNK
N_GROUPS = CHUNK // LANES
ROWS_PER_FD = VOCAB // EMBED_DIM
FROWS = EMBED_DIM * CHUNK
HROWS = FROWS // 8
N_HALF = NUM_FIELDS * 8
NBUF = 16


def _pconst(i):
    return i * (2 * NUM_FIELDS - i - 1) // 2 - i - 1


def _fam_body(x_hbm, w_hbm, out_hbm, xb, vrb, cvb, bufs, ebt, ob, sems):
    wid = lax.axis_index("s") * NC + lax.axis_index("c")
    iota = lax.iota(jnp.int32, LANES)

    def chunk_body(t, _):
        base = wid * PER_W + t * CHUNK

        pltpu.sync_copy(x_hbm.at[pl.ds(base, CHUNK)], xb)
        for f in range(NUM_FIELDS):
            fspl = jnp.full((LANES,), f, jnp.int32)
            for g in range(N_GROUPS):
                rowv = g * LANES + iota
                v = plsc.load_gather(xb, [rowv, fspl])
                cvb[pl.ds(f * CHUNK + g * LANES, LANES)] = v & 15
                vr = lax.shift_right_logical(v, 4)
                for d in range(EMBED_DIM):
                    u = f * EMBED_DIM + d
                    vrb[pl.ds(u * CHUNK + g * LANES, LANES)] = (
                        vr + u * ROWS_PER_FD)

        def fire(h, slot):
            return pltpu.async_copy(
                w_hbm.at[vrb.at[pl.ds(h * HROWS, HROWS)]],
                bufs.at[pl.ds(slot * HROWS, HROWS)], sems.at[slot])

        for s in range(NBUF):
            fire(s, s)

        def half_body(h, _):
            slot = lax.rem(h, NBUF)
            pltpu.make_async_copy(
                w_hbm.at[vrb.at[pl.ds(h * HROWS, HROWS)]],
                bufs.at[pl.ds(slot * HROWS, HROWS)], sems.at[slot]).wait()
            f = h // 8
            sbase = jnp.full((LANES,), 1, jnp.int32) * (slot * HROWS) + iota
            ebase = h * HROWS
            for g in range(N_GROUPS):
                voff = cvb[pl.ds(f * CHUNK + g * LANES, LANES)]
                for dl in range(EMBED_DIM // 8):
                    srow = sbase + (dl * CHUNK + g * LANES)
                    ev = plsc.load_gather(bufs, [srow, voff])
                    ebt[pl.ds(ebase + dl * CHUNK + g * LANES, LANES)] = ev

            @pl.when(h + NBUF < N_HALF)
            def _():
                fire(h + NBUF, slot)

            return 0

        lax.fori_loop(0, N_HALF, half_body, 0)

        def tree(prods):
            while len(prods) > 1:
                nxt = [prods[k] + prods[k + 1]
                       for k in range(0, len(prods) - 1, 2)]
                if len(prods) % 2:
                    nxt.append(prods[-1])
                prods = nxt
            return prods[0]

        def group_body(g, _):
            g16 = g * LANES
            rows = g16 + iota
            for i0 in range(0, NUM_FIELDS, 2):
                i1 = i0 + 1
                e0 = [ebt[pl.ds((i0 * EMBED_DIM + d) * CHUNK + g16, LANES)]
                      for d in range(EMBED_DIM)]
                e1 = [ebt[pl.ds((i1 * EMBED_DIM + d) * CHUNK + g16, LANES)]
                      for d in range(EMBED_DIM)]
                p01 = jnp.full((LANES,), 1, jnp.int32) * (_pconst(i0) + i1)
                plsc.store_scatter(
                    ob, [rows, p01],
                    tree([e0[d] * e1[d] for d in range(EMBED_DIM)]))
                if i1 == NUM_FIELDS - 1:
                    continue
                pc0, pc1 = _pconst(i0), _pconst(i1)

                @plsc.parallel_loop(i0 + 2, NUM_FIELDS, unroll=2)
                def j_body(j, e0=e0, e1=e1, g16=g16, rows=rows,
                           pc0=pc0, pc1=pc1):
                    jb = j * EMBED_DIM * CHUNK + g16
                    ej = [ebt[pl.ds(jb + d * CHUNK, LANES)]
                          for d in range(EMBED_DIM)]
                    acc0 = tree([e0[d] * ej[d] for d in range(EMBED_DIM)])
                    acc1 = tree([e1[d] * ej[d] for d in range(EMBED_DIM)])
                    one = jnp.full((LANES,), 1, jnp.int32)
                    plsc.store_scatter(ob, [rows, one * (pc0 + j)], acc0)
                    plsc.store_scatter(ob, [rows, one * (pc1 + j)], acc1)

            return 0

        lax.fori_loop(0, N_GROUPS, group_body, 0)

        pltpu.sync_copy(ob, out_hbm.at[pl.ds(base, CHUNK)])
        return 0

    lax.fori_loop(0, N_ITERS, chunk_body, 0)


@jax.jit
def _fam(x, w2):
    mesh = plsc.VectorSubcoreMesh(core_axis_name="c", subcore_axis_name="s")
    return pl.kernel(
        _fam_body,
        out_type=jax.ShapeDtypeStruct((BATCH, NUM_PAIRS), jnp.float32),
        mesh=mesh,
        compiler_params=pltpu.CompilerParams(
            needs_layout_passes=False, use_tc_tiling_on_sc=False),
        scratch_types=[
            pltpu.VMEM((CHUNK, NUM_FIELDS), jnp.int32),
            pltpu.VMEM((NUM_FIELDS * FROWS,), jnp.int32),
            pltpu.VMEM((NUM_FIELDS * CHUNK,), jnp.int32),
            pltpu.VMEM((NBUF * HROWS, EMBED_DIM), jnp.float32),
            pltpu.VMEM((NUM_FIELDS * EMBED_DIM * CHUNK,), jnp.float32),
            pltpu.VMEM((CHUNK, NUM_PAIRS), jnp.float32),
            pltpu.SemaphoreType.DMA((NBUF,)),
        ],
    )(x, w2)


def kernel(x, W):
    w2 = W.transpose(0, 2, 1).reshape(
        NUM_FIELDS * EMBED_DIM * ROWS_PER_FD, EMBED_DIM)
    return _fam(x.astype(jnp.int32), w2)


# speedup vs baseline: 1.0291x; 1.0291x over previous
import jax
import jax.numpy as jnp
from jax import lax
from jax.experimental import pallas as pl
from jax.experimental.pallas import tpu as pltpu
from jax.experimental.pallas import tpu_sc as plsc

NUM_FIELDS = 26
VOCAB = 100000
EMBED_DIM = 16
BATCH = 16384
NUM_PAIRS = (NUM_FIELDS * (NUM_FIELDS - 1)) // 2

_INFO = plsc.get_sparse_core_info()
NC = _INFO.num_cores
NS = _INFO.num_subcores
NW = NC * NS
LANES = _INFO.num_lanes

CHUNK = 64
PER_W = BATCH // NW
N_ITERS = PER_W // CHUNK
N_GROUPS = CHUNK // LANES
ROWS_PER_FD = VOCAB // EMBED_DIM
FROWS = EMBED_DIM * CHUNK
HROWS = FROWS // 4
N_HALF = NUM_FIELDS * 4
NBUF = 8


def _pconst(i):
    return i * (2 * NUM_FIELDS - i - 1) // 2 - i - 1


def _fam_body(x_hbm, w_hbm, out_hbm, xb, vrb, cvb, bufs, ebt, ob, sems):
    wid = lax.axis_index("s") * NC + lax.axis_index("c")
    iota = lax.iota(jnp.int32, LANES)

    def chunk_body(t, _):
        base = wid * PER_W + t * CHUNK

        pltpu.sync_copy(x_hbm.at[pl.ds(base, CHUNK)], xb)
        for f in range(NUM_FIELDS):
            fspl = jnp.full((LANES,), f, jnp.int32)
            for g in range(N_GROUPS):
                rowv = g * LANES + iota
                v = plsc.load_gather(xb, [rowv, fspl])
                cvb[pl.ds(f * CHUNK + g * LANES, LANES)] = v & 15
                vr = lax.shift_right_logical(v, 4)
                for d in range(EMBED_DIM):
                    u = f * EMBED_DIM + d
                    vrb[pl.ds(u * CHUNK + g * LANES, LANES)] = (
                        vr + u * ROWS_PER_FD)

        def fire(h, slot):
            return pltpu.async_copy(
                w_hbm.at[vrb.at[pl.ds(h * HROWS, HROWS)]],
                bufs.at[pl.ds(slot * HROWS, HROWS)], sems.at[slot])

        for s in range(NBUF):
            fire(s, s)

        def half_body(h, _):
            slot = lax.rem(h, NBUF)
            pltpu.make_async_copy(
                w_hbm.at[vrb.at[pl.ds(h * HROWS, HROWS)]],
                bufs.at[pl.ds(slot * HROWS, HROWS)], sems.at[slot]).wait()
            f = h // 4
            sbase = jnp.full((LANES,), 1, jnp.int32) * (slot * HROWS) + iota
            ebase = h * HROWS
            for g in range(N_GROUPS):
                voff = cvb[pl.ds(f * CHUNK + g * LANES, LANES)]
                for dl in range(EMBED_DIM // 4):
                    srow = sbase + (dl * CHUNK + g * LANES)
                    ev = plsc.load_gather(bufs, [srow, voff])
                    ebt[pl.ds(ebase + dl * CHUNK + g * LANES, LANES)] = ev

            @pl.when(h + NBUF < N_HALF)
            def _():
                fire(h + NBUF, slot)

            return 0

        lax.fori_loop(0, N_HALF, half_body, 0)

        def tree(prods):
            while len(prods) > 1:
                nxt = [prods[k] + prods[k + 1]
                       for k in range(0, len(prods) - 1, 2)]
                if len(prods) % 2:
                    nxt.append(prods[-1])
                prods = nxt
            return prods[0]

        def group_body(g, _):
            g16 = g * LANES
            rows = g16 + iota
            for i0 in range(0, NUM_FIELDS, 2):
                i1 = i0 + 1
                e0 = [ebt[pl.ds((i0 * EMBED_DIM + d) * CHUNK + g16, LANES)]
                      for d in range(EMBED_DIM)]
                e1 = [ebt[pl.ds((i1 * EMBED_DIM + d) * CHUNK + g16, LANES)]
                      for d in range(EMBED_DIM)]
                p01 = jnp.full((LANES,), 1, jnp.int32) * (_pconst(i0) + i1)
                plsc.store_scatter(
                    ob, [rows, p01],
                    tree([e0[d] * e1[d] for d in range(EMBED_DIM)]))
                if i1 == NUM_FIELDS - 1:
                    continue
                pc0, pc1 = _pconst(i0), _pconst(i1)

                @plsc.parallel_loop(i0 + 2, NUM_FIELDS, unroll=2)
                def j_body(j, e0=e0, e1=e1, g16=g16, rows=rows,
                           pc0=pc0, pc1=pc1):
                    jb = j * EMBED_DIM * CHUNK + g16
                    ej = [ebt[pl.ds(jb + d * CHUNK, LANES)]
                          for d in range(EMBED_DIM)]
                    acc0 = tree([e0[d] * ej[d] for d in range(EMBED_DIM)])
                    acc1 = tree([e1[d] * ej[d] for d in range(EMBED_DIM)])
                    one = jnp.full((LANES,), 1, jnp.int32)
                    plsc.store_scatter(ob, [rows, one * (pc0 + j)], acc0)
                    plsc.store_scatter(ob, [rows, one * (pc1 + j)], acc1)

            return 0

        lax.fori_loop(0, N_GROUPS, group_body, 0)

        pltpu.sync_copy(ob, out_hbm.at[pl.ds(base, CHUNK)])
        return 0

    lax.fori_loop(0, N_ITERS, chunk_body, 0)


@jax.jit
def _fam(x, w2):
    mesh = plsc.VectorSubcoreMesh(core_axis_name="c", subcore_axis_name="s")
    return pl.kernel(
        _fam_body,
        out_type=jax.ShapeDtypeStruct((BATCH, NUM_PAIRS), jnp.float32),
        mesh=mesh,
        compiler_params=pltpu.CompilerParams(
            needs_layout_passes=False, use_tc_tiling_on_sc=False),
        scratch_types=[
            pltpu.VMEM((CHUNK, NUM_FIELDS), jnp.int32),
            pltpu.VMEM((NUM_FIELDS * FROWS,), jnp.int32),
            pltpu.VMEM((NUM_FIELDS * CHUNK,), jnp.int32),
            pltpu.VMEM((NBUF * HROWS, EMBED_DIM), jnp.float32),
            pltpu.VMEM((NUM_FIELDS * EMBED_DIM * CHUNK,), jnp.float32),
            pltpu.VMEM((CHUNK, NUM_PAIRS), jnp.float32),
            pltpu.SemaphoreType.DMA((NBUF,)),
        ],
    )(x, w2)


def kernel(x, W):
    w2 = W.transpose(0, 2, 1).reshape(
        NUM_FIELDS * EMBED_DIM * ROWS_PER_FD, EMBED_DIM)
    return _fam(x.astype(jnp.int32), w2)
